# baseline (speedup 1.0000x reference)
import jax
import jax.numpy as jnp
from jax.experimental import pallas as pl
from jax.experimental.pallas import tpu as pltpu

C = 128
N = 8192
T_TILE = 512
T_HALF = T_TILE // 2
S_CHUNK = 512
TB_PER_CORE = 8
NEG_INF = float("-inf")


def _linear_kernel(w_ref, x_ref, b_ref, o_ref):
    h = jnp.dot(w_ref[...], x_ref[...], preferred_element_type=jnp.float32)
    o_ref[...] = (h + b_ref[...]).astype(jnp.bfloat16)


def _linear_t(w, x_t, b_col):
    tile = min(1024, N)
    return pl.pallas_call(
        _linear_kernel,
        out_shape=jax.ShapeDtypeStruct((C, N), jnp.bfloat16),
        grid=(N // tile,),
        in_specs=[
            pl.BlockSpec((C, C), lambda i: (0, 0)),
            pl.BlockSpec((C, tile), lambda i: (0, i)),
            pl.BlockSpec((C, 1), lambda i: (0, 0)),
        ],
        out_specs=pl.BlockSpec((C, tile), lambda i: (0, i)),
        compiler_params=pltpu.CompilerParams(
            dimension_semantics=("parallel",)),
    )(w, x_t, b_col)


def _build_bcast(h_ref, hbc_ref):
    h_blk = h_ref[...]
    for s in range(S_CHUNK):
        col = jax.lax.slice(h_blk, (0, s), (C, s + 1))
        hbc_ref[s] = jax.lax.broadcast_in_dim(col, (C, T_HALF), (0, 1))


def _accumulate(acc_l, acc_r, mask_blk, hbc_ref):
    for s in range(S_CHUNK):
        hb = hbc_ref[s]
        m = mask_blk[s:s + 1, :]
        acc_l = jnp.maximum(acc_l, hb + m[:, :T_HALF])
        acc_r = jnp.maximum(acc_r, hb + m[:, T_HALF:])
    return acc_l, acc_r


def _agg_steps(mask_ref, h_ref, hbc_ref, acc_ref, sc, tbi):
    @pl.when(tbi == 0)
    def _build():
        _build_bcast(h_ref, hbc_ref)

    @pl.when(sc == 0)
    def _init():
        acc_ref[tbi] = jnp.full((2, C, T_HALF), NEG_INF, jnp.bfloat16)

    acc_l, acc_r = _accumulate(acc_ref[tbi, 0], acc_ref[tbi, 1],
                               mask_ref[...], hbc_ref)
    acc_ref[tbi, 0] = acc_l
    acc_ref[tbi, 1] = acc_r


def _agg_lin_kernel(mask_ref, h_ref, w_ref, b_ref, o_ref, hbc_ref, acc_ref):
    sc = pl.program_id(1)
    tbi = pl.program_id(2)
    _agg_steps(mask_ref, h_ref, hbc_ref, acc_ref, sc, tbi)

    @pl.when(sc == pl.num_programs(1) - 1)
    def _finalize():
        for half in range(2):
            a = acc_ref[tbi, half]
            a = jnp.where(a > NEG_INF, a, jnp.bfloat16(0.0))
            a = jnp.maximum(a, jnp.bfloat16(0.0))
            h2 = jnp.dot(w_ref[...], a, preferred_element_type=jnp.float32)
            o_ref[:, half * T_HALF:(half + 1) * T_HALF] = (
                h2 + b_ref[...]).astype(jnp.bfloat16)


def _agg_out_kernel(mask_ref, h_ref, o_ref, hbc_ref, acc_ref):
    sc = pl.program_id(1)
    tbi = pl.program_id(2)
    _agg_steps(mask_ref, h_ref, hbc_ref, acc_ref, sc, tbi)

    @pl.when(sc == pl.num_programs(1) - 1)
    def _finalize():
        for half in range(2):
            a = acc_ref[tbi, half]
            o_ref[:, half * T_HALF:(half + 1) * T_HALF] = jnp.where(
                a > NEG_INF, a, jnp.bfloat16(0.0)).astype(jnp.float32)


def _agg_grid_specs():
    return dict(
        grid=(2, N // S_CHUNK, TB_PER_CORE),
        scratch_shapes=[
            pltpu.VMEM((S_CHUNK, C, T_HALF), jnp.bfloat16),
            pltpu.VMEM((TB_PER_CORE, 2, C, T_HALF), jnp.bfloat16),
        ],
        compiler_params=pltpu.CompilerParams(
            dimension_semantics=("parallel", "arbitrary", "arbitrary")),
    )


def _mask_spec():
    return pl.BlockSpec(
        (S_CHUNK, T_TILE),
        lambda tbo, sc, tbi: (sc, tbo * TB_PER_CORE + tbi))


def _out_spec():
    last = N // S_CHUNK - 1
    return pl.BlockSpec(
        (C, T_TILE),
        lambda tbo, sc, tbi: (
            0,
            jnp.where(sc == last, tbo * TB_PER_CORE + tbi,
                      tbo * TB_PER_CORE)))


def _agg_linear(mask_b, h_t, w, b_col):
    return pl.pallas_call(
        _agg_lin_kernel,
        out_shape=jax.ShapeDtypeStruct((C, N), jnp.bfloat16),
        in_specs=[
            _mask_spec(),
            pl.BlockSpec((C, S_CHUNK), lambda tbo, sc, tbi: (0, sc)),
            pl.BlockSpec((C, C), lambda tbo, sc, tbi: (0, 0)),
            pl.BlockSpec((C, 1), lambda tbo, sc, tbi: (0, 0)),
        ],
        out_specs=_out_spec(),
        **_agg_grid_specs(),
    )(mask_b, h_t, w, b_col)


def _agg_final(mask_b, h_t):
    return pl.pallas_call(
        _agg_out_kernel,
        out_shape=jax.ShapeDtypeStruct((C, N), jnp.float32),
        in_specs=[
            _mask_spec(),
            pl.BlockSpec((C, S_CHUNK), lambda tbo, sc, tbi: (0, sc)),
        ],
        out_specs=_out_spec(),
        **_agg_grid_specs(),
    )(mask_b, h_t)


def kernel(w1_t, b1, w2_t, b2, x, neg_mask):
    mask_b = neg_mask.T
    x_t = x.T.astype(jnp.bfloat16)
    w1 = w1_t.T
    w2 = w2_t.T
    b1_col = b1.T
    b2_col = b2.T

    h1_t = _linear_t(w1, x_t, b1_col)
    h2_t = _agg_linear(mask_b, h1_t, w2, b2_col)
    a2_t = _agg_final(mask_b, h2_t)
    return a2_t.T


# speedup vs baseline: 1.0828x; 1.0828x over previous
import jax
import jax.numpy as jnp
from jax.experimental import pallas as pl
from jax.experimental.pallas import tpu as pltpu

C = 128
N = 8192
T_TILE = 512
S_CHUNK = 256
TB_PER_CORE = 8
NEG_INF = float("-inf")


def _linear_kernel(w_ref, x_ref, b_ref, o_ref):
    h = jnp.dot(w_ref[...], x_ref[...], preferred_element_type=jnp.float32)
    o_ref[...] = (h + b_ref[...]).astype(jnp.bfloat16)


def _linear_t(w, x_t, b_col):
    tile = min(1024, N)
    return pl.pallas_call(
        _linear_kernel,
        out_shape=jax.ShapeDtypeStruct((C, N), jnp.bfloat16),
        grid=(N // tile,),
        in_specs=[
            pl.BlockSpec((C, C), lambda i: (0, 0)),
            pl.BlockSpec((C, tile), lambda i: (0, i)),
            pl.BlockSpec((C, 1), lambda i: (0, 0)),
        ],
        out_specs=pl.BlockSpec((C, tile), lambda i: (0, i)),
        compiler_params=pltpu.CompilerParams(
            dimension_semantics=("parallel",)),
    )(w, x_t, b_col)


def _build_bcast(h_ref, hbc_ref):
    h_blk = h_ref[...]
    for s in range(S_CHUNK):
        col = jax.lax.slice(h_blk, (0, s), (C, s + 1))
        hbc_ref[s] = jax.lax.broadcast_in_dim(col, (C, T_TILE), (0, 1))


def _accumulate(acc, mask_blk, hbc_ref):
    for s in range(S_CHUNK):
        acc = jnp.maximum(acc, hbc_ref[s] + mask_blk[s:s + 1, :])
    return acc


def _agg_steps(mask_ref, h_ref, hbc_ref, acc_ref, sc, tbi):
    @pl.when(tbi == 0)
    def _build():
        _build_bcast(h_ref, hbc_ref)

    @pl.when(sc == 0)
    def _init():
        acc_ref[tbi] = jnp.full((C, T_TILE), NEG_INF, jnp.bfloat16)

    acc_ref[tbi] = _accumulate(acc_ref[tbi], mask_ref[...], hbc_ref)


def _agg_lin_kernel(mask_ref, h_ref, w_ref, b_ref, o_ref, hbc_ref, acc_ref):
    sc = pl.program_id(1)
    tbi = pl.program_id(2)
    _agg_steps(mask_ref, h_ref, hbc_ref, acc_ref, sc, tbi)

    @pl.when(sc == pl.num_programs(1) - 1)
    def _finalize():
        a = acc_ref[tbi]
        a = jnp.where(a > NEG_INF, a, jnp.bfloat16(0.0))
        a = jnp.maximum(a, jnp.bfloat16(0.0))
        h2 = jnp.dot(w_ref[...], a, preferred_element_type=jnp.float32)
        o_ref[...] = (h2 + b_ref[...]).astype(jnp.bfloat16)


def _agg_out_kernel(mask_ref, h_ref, o_ref, hbc_ref, acc_ref):
    sc = pl.program_id(1)
    tbi = pl.program_id(2)
    _agg_steps(mask_ref, h_ref, hbc_ref, acc_ref, sc, tbi)

    @pl.when(sc == pl.num_programs(1) - 1)
    def _finalize():
        a = acc_ref[tbi]
        o_ref[...] = jnp.where(a > NEG_INF, a, jnp.bfloat16(0.0)
                               ).astype(jnp.float32)


def _agg_grid_specs():
    return dict(
        grid=(2, N // S_CHUNK, TB_PER_CORE),
        scratch_shapes=[
            pltpu.VMEM((S_CHUNK, C, T_TILE), jnp.bfloat16),
            pltpu.VMEM((TB_PER_CORE, C, T_TILE), jnp.bfloat16),
        ],
        compiler_params=pltpu.CompilerParams(
            dimension_semantics=("parallel", "arbitrary", "arbitrary")),
    )


def _mask_spec():
    return pl.BlockSpec(
        (S_CHUNK, T_TILE),
        lambda tbo, sc, tbi: (sc, tbo * TB_PER_CORE + tbi))


def _out_spec():
    last = N // S_CHUNK - 1
    return pl.BlockSpec(
        (C, T_TILE),
        lambda tbo, sc, tbi: (
            0,
            jnp.where(sc == last, tbo * TB_PER_CORE + tbi,
                      tbo * TB_PER_CORE)))


def _agg_linear(mask_b, h_t, w, b_col):
    return pl.pallas_call(
        _agg_lin_kernel,
        out_shape=jax.ShapeDtypeStruct((C, N), jnp.bfloat16),
        in_specs=[
            _mask_spec(),
            pl.BlockSpec((C, S_CHUNK), lambda tbo, sc, tbi: (0, sc)),
            pl.BlockSpec((C, C), lambda tbo, sc, tbi: (0, 0)),
            pl.BlockSpec((C, 1), lambda tbo, sc, tbi: (0, 0)),
        ],
        out_specs=_out_spec(),
        **_agg_grid_specs(),
    )(mask_b, h_t, w, b_col)


def _agg_final(mask_b, h_t):
    return pl.pallas_call(
        _agg_out_kernel,
        out_shape=jax.ShapeDtypeStruct((C, N), jnp.float32),
        in_specs=[
            _mask_spec(),
            pl.BlockSpec((C, S_CHUNK), lambda tbo, sc, tbi: (0, sc)),
        ],
        out_specs=_out_spec(),
        **_agg_grid_specs(),
    )(mask_b, h_t)


def kernel(w1_t, b1, w2_t, b2, x, neg_mask):
    mask_b = neg_mask.T
    x_t = x.T.astype(jnp.bfloat16)
    w1 = w1_t.T
    w2 = w2_t.T
    b1_col = b1.T
    b2_col = b2.T

    h1_t = _linear_t(w1, x_t, b1_col)
    h2_t = _agg_linear(mask_b, h1_t, w2, b2_col)
    a2_t = _agg_final(mask_b, h2_t)
    return a2_t.T
